# baseline (speedup 1.0000x reference)
import dataclasses
import functools

import jax
import jax.numpy as jnp
from jax import lax
from jax.experimental import pallas as pl
from jax.experimental.pallas import tpu as pltpu
from jax.experimental.pallas import tpu_sc as plsc

B, N, C, M = 8, 16384, 256, 2048
NC, NS = 2, 16
NW = NC * NS
TPB = NW // B
CPT = C // TPB
MPT = M // TPB
L = 16
ROWS = 8
SLABS = CPT // ROWS
NQ = 4
QW = N // NQ
NT = SLABS * NQ
XCH = MPT * 3 // 128


def _compiler_params():
    cp = pltpu.CompilerParams()
    fields = pltpu.CompilerParams.__dataclass_fields__
    if "needs_layout_passes" in fields:
        cp = dataclasses.replace(cp, needs_layout_passes=False)
    return cp


def _sc_gather(xyz_flat, feat, inds3):
    mesh = plsc.VectorSubcoreMesh(core_axis_name="c", subcore_axis_name="s")

    @functools.partial(
        pl.kernel,
        compiler_params=_compiler_params(),
        out_type=(
            jax.ShapeDtypeStruct((B, TPB, XCH, 128), jnp.float32),
            jax.ShapeDtypeStruct((B, C, M), jnp.float32),
        ),
        mesh=mesh,
        scratch_types=[
            pltpu.VMEM((M // 128, 128), jnp.int32),
            pltpu.VMEM((ROWS, QW), jnp.float32),
            pltpu.VMEM((ROWS, QW), jnp.float32),
            pltpu.VMEM((ROWS, M), jnp.float32),
            pltpu.VMEM((ROWS, M), jnp.float32),
            pltpu.VMEM((XCH, 128), jnp.int32),
            pltpu.VMEM((XCH, 128), jnp.float32),
            pltpu.SemaphoreType.DMA,
            pltpu.SemaphoreType.DMA,
            pltpu.SemaphoreType.DMA,
            pltpu.SemaphoreType.DMA,
            pltpu.SemaphoreType.DMA,
        ],
    )
    def k(xyz_hbm, feat_hbm, inds_hbm, oxyz_hbm, ofeat_hbm,
          inds_v, buf_a, buf_b, obuf_a, obuf_b, widx_v, oxyz_v,
          isem_a, isem_b, osem_a, osem_b, xsem):
        wid = lax.axis_index("c") * NS + lax.axis_index("s")
        b = wid // TPB
        q_tile = wid % TPB
        cbase = q_tile * CPT
        bufs = (buf_a, buf_b)
        obufs = (obuf_a, obuf_b)
        isems = (isem_a, isem_b)
        osems = (osem_a, osem_b)
        iota = lax.iota(jnp.int32, L)
        rsplat = [jnp.full((L,), r, jnp.int32) for r in range(ROWS)]

        def src(t):
            c0 = pl.multiple_of(cbase + (t // NQ) * ROWS, ROWS)
            n0 = pl.multiple_of((t % NQ) * QW, QW)
            return feat_hbm.at[b, pl.ds(c0, ROWS), pl.ds(n0, QW)]

        def odst(slab):
            c0 = pl.multiple_of(cbase + slab * ROWS, ROWS)
            return ofeat_hbm.at[b, pl.ds(c0, ROWS), :]

        pltpu.sync_copy(inds_hbm.at[b], inds_v)

        pltpu.async_copy(src(0), buf_a, isem_a)
        pltpu.async_copy(src(1), buf_b, isem_b)

        mbase = q_tile * MPT

        @pl.loop(0, MPT, step=L)
        def _(ml):
            mg = mbase + ml
            idx = inds_v[mg // 128, pl.ds(mg % 128, L)]
            g = idx * 3 + (b * N * 3)
            p0 = (ml + iota) * 3
            for j in range(3):
                p = p0 + j
                plsc.store_scatter(widx_v, [p // 128, p % 128], g + j)

        for ch in range(XCH):
            pltpu.async_copy(xyz_hbm.at[widx_v.at[ch]], oxyz_v.at[ch], xsem)
        for ch in range(XCH):
            pltpu.make_async_copy(
                xyz_hbm.at[widx_v.at[ch]], oxyz_v.at[ch], xsem).wait()
        pltpu.sync_copy(oxyz_v, oxyz_hbm.at[b, q_tile])

        @pl.loop(0, SLABS, step=2)
        def _(si):
            for sp in range(2):
                slab = si + sp
                obuf = obufs[sp]
                osem = osems[sp]

                @pl.when(slab >= 2)
                def _():
                    pltpu.make_async_copy(obuf, odst(slab - 2), osem).wait()

                for q in range(NQ):
                    buf = bufs[q % 2]
                    isem = isems[q % 2]
                    t = slab * NQ + q
                    pltpu.make_async_copy(src(t), buf, isem).wait()

                    @pl.loop(0, M // 128)
                    def _(ir):
                        for wc in range(8):
                            idx = inds_v[ir, pl.ds(wc * L, L)]
                            mask = (idx >> 12) == q
                            nq = idx & (QW - 1)
                            mvec = ir * 128 + wc * L + iota
                            for r in range(ROWS):
                                v = plsc.load_gather(
                                    buf, [rsplat[r], nq], mask=mask)
                                plsc.store_scatter(
                                    obuf, [rsplat[r], mvec], v, mask=mask)

                    @pl.when(t + 2 < NT)
                    def _():
                        pltpu.async_copy(src(t + 2), buf, isem)

                pltpu.async_copy(obuf, odst(slab), osem)

        pltpu.make_async_copy(obuf_a, odst(SLABS - 2), osem_a).wait()
        pltpu.make_async_copy(obuf_b, odst(SLABS - 1), osem_b).wait()

    return k(xyz_flat, feat, inds3)


def kernel(xyz, features, sample_inds):
    inds3 = sample_inds.astype(jnp.int32).reshape(B, M // 128, 128)
    xyz_flat = xyz.reshape(B * N * 3)
    oxyz, new_features = _sc_gather(xyz_flat, features, inds3)
    new_xyz = oxyz.reshape(B, M, 3)
    return (new_xyz, new_features, sample_inds)


# speedup vs baseline: 1.5177x; 1.5177x over previous
import dataclasses
import functools

import jax
import jax.numpy as jnp
from jax import lax
from jax.experimental import pallas as pl
from jax.experimental.pallas import tpu as pltpu
from jax.experimental.pallas import tpu_sc as plsc

B, N, C, M = 8, 16384, 256, 2048
NC, NS = 2, 16
NW = NC * NS
TPB = NW // B
CPT = C // TPB
MPT = M // TPB
L = 16
ROWS = 8
SLABS = CPT // ROWS
NQ = 4
QW = N // NQ
NT = SLABS * NQ
XCH = MPT * 3 // 128


def _compiler_params():
    cp = pltpu.CompilerParams()
    fields = pltpu.CompilerParams.__dataclass_fields__
    if "needs_layout_passes" in fields:
        cp = dataclasses.replace(cp, needs_layout_passes=False)
    return cp


def _sc_gather(xyz_flat, feat, inds3):
    mesh = plsc.VectorSubcoreMesh(core_axis_name="c", subcore_axis_name="s")

    @functools.partial(
        pl.kernel,
        compiler_params=_compiler_params(),
        out_type=(
            jax.ShapeDtypeStruct((B, TPB, XCH, 128), jnp.float32),
            jax.ShapeDtypeStruct((B, C, M), jnp.float32),
        ),
        mesh=mesh,
        scratch_types=[
            pltpu.VMEM((M // 128, 128), jnp.int32),
            pltpu.VMEM((ROWS, QW), jnp.float32),
            pltpu.VMEM((ROWS, QW), jnp.float32),
            pltpu.VMEM((ROWS, M), jnp.float32),
            pltpu.VMEM((ROWS, M), jnp.float32),
            pltpu.VMEM((XCH, 128), jnp.int32),
            pltpu.VMEM((XCH, 128), jnp.float32),
            pltpu.VMEM((NQ, M), jnp.int32),
            pltpu.SemaphoreType.DMA,
            pltpu.SemaphoreType.DMA,
            pltpu.SemaphoreType.DMA,
            pltpu.SemaphoreType.DMA,
            pltpu.SemaphoreType.DMA,
        ],
    )
    def k(xyz_hbm, feat_hbm, inds_hbm, oxyz_hbm, ofeat_hbm,
          inds_v, buf_a, buf_b, obuf_a, obuf_b, widx_v, oxyz_v, elist_v,
          isem_a, isem_b, osem_a, osem_b, xsem):
        wid = lax.axis_index("c") * NS + lax.axis_index("s")
        b = wid // TPB
        q_tile = wid % TPB
        cbase = q_tile * CPT
        bufs = (buf_a, buf_b)
        obufs = (obuf_a, obuf_b)
        isems = (isem_a, isem_b)
        osems = (osem_a, osem_b)
        iota = lax.iota(jnp.int32, L)
        rsplat = [jnp.full((L,), r, jnp.int32) for r in range(ROWS)]

        def src(t):
            c0 = pl.multiple_of(cbase + (t // NQ) * ROWS, ROWS)
            n0 = pl.multiple_of((t % NQ) * QW, QW)
            return feat_hbm.at[b, pl.ds(c0, ROWS), pl.ds(n0, QW)]

        def odst(slab):
            c0 = pl.multiple_of(cbase + slab * ROWS, ROWS)
            return ofeat_hbm.at[b, pl.ds(c0, ROWS), :]

        pltpu.sync_copy(inds_hbm.at[b], inds_v)

        pltpu.async_copy(src(0), buf_a, isem_a)
        pltpu.async_copy(src(1), buf_b, isem_b)

        mbase = q_tile * MPT

        @pl.loop(0, MPT, step=L)
        def _(ml):
            mg = mbase + ml
            idx = inds_v[mg // 128, pl.ds(mg % 128, L)]
            g = idx * 3 + (b * N * 3)
            p0 = (ml + iota) * 3
            for j in range(3):
                p = p0 + j
                plsc.store_scatter(widx_v, [p // 128, p % 128], g + j)

        for ch in range(XCH):
            pltpu.async_copy(xyz_hbm.at[widx_v.at[ch]], oxyz_v.at[ch], xsem)
        for ch in range(XCH):
            pltpu.make_async_copy(
                xyz_hbm.at[widx_v.at[ch]], oxyz_v.at[ch], xsem).wait()
        pltpu.sync_copy(oxyz_v, oxyz_hbm.at[b, q_tile])

        qsplat = [jnp.full((L,), q, jnp.int32) for q in range(NQ)]

        def compact_body(w, cnts):
            row = w // 8
            col = pl.multiple_of((w % 8) * L, L)
            idx = inds_v[row, pl.ds(col, L)]
            mvec = w * L + iota
            new = []
            for q in range(NQ):
                mask = (idx >> 12) == q
                mi = mask.astype(jnp.int32)
                pos = cnts[q] + plsc.cumsum(mi) - 1
                e = (mvec << 12) | (idx & (QW - 1))
                plsc.store_scatter(elist_v, [qsplat[q], pos], e, mask=mask)
                new.append(cnts[q] + jnp.sum(mi))
            return tuple(new)

        cnts = lax.fori_loop(
            0, M // L, compact_body,
            tuple(jnp.int32(0) for _ in range(NQ)))
        nwins = [(cnts[q] + (L - 1)) // L for q in range(NQ)]

        @pl.loop(0, SLABS, step=2)
        def _(si):
            for sp in range(2):
                slab = si + sp
                obuf = obufs[sp]
                osem = osems[sp]

                @pl.when(slab >= 2)
                def _():
                    pltpu.make_async_copy(obuf, odst(slab - 2), osem).wait()

                for q in range(NQ):
                    buf = bufs[q % 2]
                    isem = isems[q % 2]
                    t = slab * NQ + q
                    pltpu.make_async_copy(src(t), buf, isem).wait()

                    @pl.loop(0, nwins[q])
                    def _(jw):
                        j = pl.multiple_of(jw * L, L)
                        e = elist_v[q, pl.ds(j, L)]
                        nq = e & (QW - 1)
                        dest = e >> 12
                        maskw = (j + iota) < cnts[q]
                        for r in range(ROWS):
                            v = plsc.load_gather(buf, [rsplat[r], nq])
                            plsc.store_scatter(
                                obuf, [rsplat[r], dest], v, mask=maskw)

                    @pl.when(t + 2 < NT)
                    def _():
                        pltpu.async_copy(src(t + 2), buf, isem)

                pltpu.async_copy(obuf, odst(slab), osem)

        pltpu.make_async_copy(obuf_a, odst(SLABS - 2), osem_a).wait()
        pltpu.make_async_copy(obuf_b, odst(SLABS - 1), osem_b).wait()

    return k(xyz_flat, feat, inds3)


def kernel(xyz, features, sample_inds):
    inds3 = sample_inds.astype(jnp.int32).reshape(B, M // 128, 128)
    xyz_flat = xyz.reshape(B * N * 3)
    oxyz, new_features = _sc_gather(xyz_flat, features, inds3)
    new_xyz = oxyz.reshape(B, M, 3)
    return (new_xyz, new_features, sample_inds)


# speedup vs baseline: 1.5519x; 1.0225x over previous
import dataclasses
import functools

import jax
import jax.numpy as jnp
from jax import lax
from jax.experimental import pallas as pl
from jax.experimental.pallas import tpu as pltpu
from jax.experimental.pallas import tpu_sc as plsc

B, N, C, M = 8, 16384, 256, 2048
NC, NS = 2, 16
NW = NC * NS
TPB = NW // B
CPT = C // TPB
MPT = M // TPB
L = 16
ROWS = 8
SLABS = CPT // ROWS
NQ = 4
QW = N // NQ
NT = SLABS * NQ
XCH = MPT * 3 // 128


def _compiler_params():
    cp = pltpu.CompilerParams()
    fields = pltpu.CompilerParams.__dataclass_fields__
    if "needs_layout_passes" in fields:
        cp = dataclasses.replace(cp, needs_layout_passes=False)
    return cp


def _mesh():
    return plsc.VectorSubcoreMesh(core_axis_name="c", subcore_axis_name="s")


def _sc_gather_features(feat, inds3):
    @functools.partial(
        pl.kernel,
        compiler_params=_compiler_params(),
        out_type=jax.ShapeDtypeStruct((B, C, M), jnp.float32),
        mesh=_mesh(),
        scratch_types=[
            pltpu.VMEM((M // 128, 128), jnp.int32),
            pltpu.VMEM((ROWS, QW), jnp.float32),
            pltpu.VMEM((ROWS, QW), jnp.float32),
            pltpu.VMEM((ROWS, M), jnp.float32),
            pltpu.VMEM((ROWS, M), jnp.float32),
            pltpu.VMEM((NQ, M), jnp.int32),
            pltpu.SemaphoreType.DMA,
            pltpu.SemaphoreType.DMA,
            pltpu.SemaphoreType.DMA,
            pltpu.SemaphoreType.DMA,
        ],
    )
    def k(feat_hbm, inds_hbm, ofeat_hbm,
          inds_v, buf_a, buf_b, obuf_a, obuf_b, elist_v,
          isem_a, isem_b, osem_a, osem_b):
        wid = lax.axis_index("c") * NS + lax.axis_index("s")
        b = wid // TPB
        q_tile = wid % TPB
        cbase = q_tile * CPT
        bufs = (buf_a, buf_b)
        obufs = (obuf_a, obuf_b)
        isems = (isem_a, isem_b)
        osems = (osem_a, osem_b)
        iota = lax.iota(jnp.int32, L)
        rsplat = [jnp.full((L,), r, jnp.int32) for r in range(ROWS)]
        qsplat = [jnp.full((L,), q, jnp.int32) for q in range(NQ)]

        def src(t):
            c0 = pl.multiple_of(cbase + (t // NQ) * ROWS, ROWS)
            n0 = pl.multiple_of((t % NQ) * QW, QW)
            return feat_hbm.at[b, pl.ds(c0, ROWS), pl.ds(n0, QW)]

        def odst(slab):
            c0 = pl.multiple_of(cbase + slab * ROWS, ROWS)
            return ofeat_hbm.at[b, pl.ds(c0, ROWS), :]

        pltpu.sync_copy(inds_hbm.at[b], inds_v)

        pltpu.async_copy(src(0), buf_a, isem_a)
        pltpu.async_copy(src(1), buf_b, isem_b)

        def compact_body(w, cnts):
            row = w // 8
            col = pl.multiple_of((w % 8) * L, L)
            idx = inds_v[row, pl.ds(col, L)]
            mvec = w * L + iota
            new = []
            for q in range(NQ):
                mask = (idx >> 12) == q
                mi = mask.astype(jnp.int32)
                pos = cnts[q] + plsc.cumsum(mi) - 1
                e = (mvec << 12) | (idx & (QW - 1))
                plsc.store_scatter(elist_v, [qsplat[q], pos], e, mask=mask)
                new.append(cnts[q] + jnp.sum(mi))
            return tuple(new)

        cnts = lax.fori_loop(
            0, M // L, compact_body,
            tuple(jnp.int32(0) for _ in range(NQ)))
        nwins = [(cnts[q] + (L - 1)) // L for q in range(NQ)]

        @pl.loop(0, SLABS, step=2)
        def _(si):
            for sp in range(2):
                slab = si + sp
                obuf = obufs[sp]
                osem = osems[sp]

                @pl.when(slab >= 2)
                def _():
                    pltpu.make_async_copy(obuf, odst(slab - 2), osem).wait()

                for q in range(NQ):
                    buf = bufs[q % 2]
                    isem = isems[q % 2]
                    t = slab * NQ + q
                    pltpu.make_async_copy(src(t), buf, isem).wait()

                    @pl.loop(0, nwins[q])
                    def _(jw):
                        j = pl.multiple_of(jw * L, L)
                        e = elist_v[q, pl.ds(j, L)]
                        nq = e & (QW - 1)
                        dest = e >> 12
                        maskw = (j + iota) < cnts[q]
                        for r in range(ROWS):
                            v = plsc.load_gather(buf, [rsplat[r], nq])
                            plsc.store_scatter(
                                obuf, [rsplat[r], dest], v, mask=maskw)

                    @pl.when(t + 2 < NT)
                    def _():
                        pltpu.async_copy(src(t + 2), buf, isem)

                pltpu.async_copy(obuf, odst(slab), osem)

        pltpu.make_async_copy(obuf_a, odst(SLABS - 2), osem_a).wait()
        pltpu.make_async_copy(obuf_b, odst(SLABS - 1), osem_b).wait()

    return k(feat, inds3)


def _sc_gather_xyz(xyz_flat, inds3):
    @functools.partial(
        pl.kernel,
        compiler_params=_compiler_params(),
        out_type=jax.ShapeDtypeStruct((B, TPB, XCH, 128), jnp.float32),
        mesh=_mesh(),
        scratch_types=[
            pltpu.VMEM((M // 128, 128), jnp.int32),
            pltpu.VMEM((XCH, 128), jnp.int32),
            pltpu.VMEM((XCH, 128), jnp.float32),
            pltpu.SemaphoreType.DMA,
        ],
    )
    def k(xyz_hbm, inds_hbm, oxyz_hbm, inds_v, widx_v, oxyz_v, xsem):
        wid = lax.axis_index("c") * NS + lax.axis_index("s")
        b = wid // TPB
        q_tile = wid % TPB
        iota = lax.iota(jnp.int32, L)

        pltpu.sync_copy(inds_hbm.at[b], inds_v)
        mbase = q_tile * MPT

        @pl.loop(0, MPT, step=L)
        def _(ml):
            mg = mbase + ml
            idx = inds_v[mg // 128, pl.ds(mg % 128, L)]
            g = idx * 3 + (b * N * 3)
            p0 = (ml + iota) * 3
            for j in range(3):
                p = p0 + j
                plsc.store_scatter(widx_v, [p // 128, p % 128], g + j)

        for ch in range(XCH):
            pltpu.async_copy(xyz_hbm.at[widx_v.at[ch]], oxyz_v.at[ch], xsem)
        for ch in range(XCH):
            pltpu.make_async_copy(
                xyz_hbm.at[widx_v.at[ch]], oxyz_v.at[ch], xsem).wait()
        pltpu.sync_copy(oxyz_v, oxyz_hbm.at[b, q_tile])

    return k(xyz_flat, inds3)


def kernel(xyz, features, sample_inds):
    inds3 = sample_inds.astype(jnp.int32).reshape(B, M // 128, 128)
    new_features = _sc_gather_features(features, inds3)
    xyz_flat = xyz.reshape(B * N * 3)
    oxyz = _sc_gather_xyz(xyz_flat, inds3)
    new_xyz = oxyz.reshape(B, M, 3)
    return (new_xyz, new_features, sample_inds)


# speedup vs baseline: 1.9513x; 1.2574x over previous
import dataclasses
import functools

import jax
import jax.numpy as jnp
from jax import lax
from jax.experimental import pallas as pl
from jax.experimental.pallas import tpu as pltpu
from jax.experimental.pallas import tpu_sc as plsc

B, N, C, M = 8, 16384, 256, 2048
NC, NS = 2, 16
NW = NC * NS
TPB = NW // B
CPT = C // TPB
MPT = M // TPB
L = 16
ROWS = 8
SLABS = CPT // ROWS
NQ = 4
QW = N // NQ
NT = SLABS * NQ
XCH = MPT * 3 // 128


def _compiler_params():
    cp = pltpu.CompilerParams()
    fields = pltpu.CompilerParams.__dataclass_fields__
    if "needs_layout_passes" in fields:
        cp = dataclasses.replace(cp, needs_layout_passes=False)
    return cp


def _mesh():
    return plsc.VectorSubcoreMesh(core_axis_name="c", subcore_axis_name="s")


def _sc_gather_features(feat, inds3):
    @functools.partial(
        pl.kernel,
        compiler_params=_compiler_params(),
        out_type=jax.ShapeDtypeStruct((B, C, M), jnp.float32),
        mesh=_mesh(),
        scratch_types=[
            pltpu.VMEM((M // 128, 128), jnp.int32),
            pltpu.VMEM((ROWS, QW), jnp.float32),
            pltpu.VMEM((ROWS, QW), jnp.float32),
            pltpu.VMEM((ROWS, M), jnp.float32),
            pltpu.VMEM((ROWS, M), jnp.float32),
            pltpu.VMEM((NQ, M), jnp.int32),
            pltpu.SemaphoreType.DMA,
            pltpu.SemaphoreType.DMA,
            pltpu.SemaphoreType.DMA,
            pltpu.SemaphoreType.DMA,
        ],
    )
    def k(feat_hbm, inds_hbm, ofeat_hbm,
          inds_v, buf_a, buf_b, obuf_a, obuf_b, elist_v,
          isem_a, isem_b, osem_a, osem_b):
        wid = lax.axis_index("c") * NS + lax.axis_index("s")
        b = wid // TPB
        q_tile = wid % TPB
        cbase = q_tile * CPT
        bufs = (buf_a, buf_b)
        obufs = (obuf_a, obuf_b)
        isems = (isem_a, isem_b)
        osems = (osem_a, osem_b)
        iota = lax.iota(jnp.int32, L)
        rsplat = [jnp.full((L,), r, jnp.int32) for r in range(ROWS)]
        qsplat = [jnp.full((L,), q, jnp.int32) for q in range(NQ)]

        def src(t):
            c0 = pl.multiple_of(cbase + (t // NQ) * ROWS, ROWS)
            n0 = pl.multiple_of((t % NQ) * QW, QW)
            return feat_hbm.at[b, pl.ds(c0, ROWS), pl.ds(n0, QW)]

        def odst(slab):
            c0 = pl.multiple_of(cbase + slab * ROWS, ROWS)
            return ofeat_hbm.at[b, pl.ds(c0, ROWS), :]

        pltpu.sync_copy(inds_hbm.at[b], inds_v)

        pltpu.async_copy(src(0), buf_a, isem_a)
        pltpu.async_copy(src(1), buf_b, isem_b)

        def compact_body(w, cnts):
            row = w // 8
            col = pl.multiple_of((w % 8) * L, L)
            idx = inds_v[row, pl.ds(col, L)]
            mvec = w * L + iota
            new = []
            for q in range(NQ):
                mask = (idx >> 12) == q
                mi = mask.astype(jnp.int32)
                pos = cnts[q] + plsc.cumsum(mi) - 1
                e = (mvec << 12) | (idx & (QW - 1))
                plsc.store_scatter(elist_v, [qsplat[q], pos], e, mask=mask)
                new.append(cnts[q] + jnp.sum(mi))
            return tuple(new)

        cnts = lax.fori_loop(
            0, M // L, compact_body,
            tuple(jnp.int32(0) for _ in range(NQ)))
        nwins = [(cnts[q] + (L - 1)) // L for q in range(NQ)]

        @pl.loop(0, SLABS, step=2)
        def _(si):
            for sp in range(2):
                slab = si + sp
                obuf = obufs[sp]
                osem = osems[sp]

                @pl.when(slab >= 2)
                def _():
                    pltpu.make_async_copy(obuf, odst(slab - 2), osem).wait()

                for q in range(NQ):
                    buf = bufs[q % 2]
                    isem = isems[q % 2]
                    t = slab * NQ + q
                    pltpu.make_async_copy(src(t), buf, isem).wait()

                    @pl.loop(0, nwins[q])
                    def _(jw):
                        j = pl.multiple_of(jw * L, L)
                        e = elist_v[q, pl.ds(j, L)]
                        nq = e & (QW - 1)
                        dest = e >> 12
                        maskw = (j + iota) < cnts[q]
                        for r in range(ROWS):
                            v = plsc.load_gather(buf, [rsplat[r], nq])
                            plsc.store_scatter(
                                obuf, [rsplat[r], dest], v, mask=maskw)

                    @pl.when(t + 2 < NT)
                    def _():
                        pltpu.async_copy(src(t + 2), buf, isem)

                pltpu.async_copy(obuf, odst(slab), osem)

        pltpu.make_async_copy(obuf_a, odst(SLABS - 2), osem_a).wait()
        pltpu.make_async_copy(obuf_b, odst(SLABS - 1), osem_b).wait()

    return k(feat, inds3)


def _sc_gather_xyz(xyz_flat, inds3):
    @functools.partial(
        pl.kernel,
        compiler_params=_compiler_params(),
        out_type=jax.ShapeDtypeStruct((B, TPB, XCH, 128), jnp.float32),
        mesh=_mesh(),
        scratch_types=[
            pltpu.VMEM((M // 128, 128), jnp.int32),
            pltpu.VMEM((XCH, 128), jnp.int32),
            pltpu.VMEM((XCH, 128), jnp.float32),
            pltpu.SemaphoreType.DMA,
        ],
    )
    def k(xyz_hbm, inds_hbm, oxyz_hbm, inds_v, widx_v, oxyz_v, xsem):
        wid = lax.axis_index("c") * NS + lax.axis_index("s")
        b = wid // TPB
        q_tile = wid % TPB
        iota = lax.iota(jnp.int32, L)

        pltpu.sync_copy(inds_hbm.at[b], inds_v)
        mbase = q_tile * MPT

        @pl.loop(0, MPT, step=L)
        def _(ml):
            mg = mbase + ml
            idx = inds_v[mg // 128, pl.ds(mg % 128, L)]
            g = idx * 3 + (b * N * 3)
            p0 = (ml + iota) * 3
            for j in range(3):
                p = p0 + j
                plsc.store_scatter(widx_v, [p // 128, p % 128], g + j)

        for ch in range(XCH):
            pltpu.async_copy(xyz_hbm.at[widx_v.at[ch]], oxyz_v.at[ch], xsem)
        for ch in range(XCH):
            pltpu.make_async_copy(
                xyz_hbm.at[widx_v.at[ch]], oxyz_v.at[ch], xsem).wait()
        pltpu.sync_copy(oxyz_v, oxyz_hbm.at[b, q_tile])

    return k(xyz_flat, inds3)


def kernel(xyz, features, sample_inds):
    inds3 = sample_inds.astype(jnp.int32).reshape(B, M // 128, 128)
    new_features = _sc_gather_features(features, inds3)
    xyz_flat = xyz.reshape(B * N * 3)
    inds3x, _ = lax.optimization_barrier((inds3, new_features))
    oxyz = _sc_gather_xyz(xyz_flat, inds3x)
    new_xyz = oxyz.reshape(B, M, 3)
    return (new_xyz, new_features, sample_inds)


# speedup vs baseline: 2.6434x; 1.3547x over previous
import dataclasses
import functools

import jax
import jax.numpy as jnp
from jax import lax
from jax.experimental import pallas as pl
from jax.experimental.pallas import tpu as pltpu
from jax.experimental.pallas import tpu_sc as plsc

B, N, C, M = 8, 16384, 256, 2048
NC, NS = 2, 16
NW = NC * NS
TPB = NW // B
CPT = C // TPB
MPT = M // TPB
L = 16
ROWS = 8
SLABS = CPT // ROWS
NQ = 4
QW = N // NQ
NT = SLABS * NQ
XCH = MPT * 3 // 128


def _compiler_params():
    cp = pltpu.CompilerParams()
    fields = pltpu.CompilerParams.__dataclass_fields__
    if "needs_layout_passes" in fields:
        cp = dataclasses.replace(cp, needs_layout_passes=False)
    return cp


def _mesh():
    return plsc.VectorSubcoreMesh(core_axis_name="c", subcore_axis_name="s")


def _sc_gather_features(feat, inds3):
    @functools.partial(
        pl.kernel,
        compiler_params=_compiler_params(),
        out_type=jax.ShapeDtypeStruct((B, C, M), jnp.float32),
        mesh=_mesh(),
        scratch_types=[
            pltpu.VMEM((M // 128, 128), jnp.int32),
            pltpu.VMEM((ROWS, QW), jnp.float32),
            pltpu.VMEM((ROWS, QW), jnp.float32),
            pltpu.VMEM((ROWS, M), jnp.float32),
            pltpu.VMEM((ROWS, M), jnp.float32),
            pltpu.VMEM((NQ, M), jnp.int32),
            pltpu.SemaphoreType.DMA,
            pltpu.SemaphoreType.DMA,
            pltpu.SemaphoreType.DMA,
            pltpu.SemaphoreType.DMA,
        ],
    )
    def k(feat_hbm, inds_hbm, ofeat_hbm,
          inds_v, buf_a, buf_b, obuf_a, obuf_b, elist_v,
          isem_a, isem_b, osem_a, osem_b):
        wid = lax.axis_index("c") * NS + lax.axis_index("s")
        b = wid // TPB
        q_tile = wid % TPB
        cbase = q_tile * CPT
        bufs = (buf_a, buf_b)
        obufs = (obuf_a, obuf_b)
        isems = (isem_a, isem_b)
        osems = (osem_a, osem_b)
        iota = lax.iota(jnp.int32, L)
        rsplat = [jnp.full((L,), r, jnp.int32) for r in range(ROWS)]
        qsplat = [jnp.full((L,), q, jnp.int32) for q in range(NQ)]

        def src(t):
            c0 = pl.multiple_of(cbase + (t // NQ) * ROWS, ROWS)
            n0 = pl.multiple_of((t % NQ) * QW, QW)
            return feat_hbm.at[b, pl.ds(c0, ROWS), pl.ds(n0, QW)]

        def odst(slab):
            c0 = pl.multiple_of(cbase + slab * ROWS, ROWS)
            return ofeat_hbm.at[b, pl.ds(c0, ROWS), :]

        pltpu.sync_copy(inds_hbm.at[b], inds_v)

        pltpu.async_copy(src(0), buf_a, isem_a)
        pltpu.async_copy(src(1), buf_b, isem_b)

        def compact_body(w, cnts):
            row = w // 8
            col = pl.multiple_of((w % 8) * L, L)
            idx = inds_v[row, pl.ds(col, L)]
            mvec = w * L + iota
            new = []
            for q in range(NQ):
                mask = (idx >> 12) == q
                mi = mask.astype(jnp.int32)
                pos = cnts[q] + plsc.cumsum(mi) - 1
                e = (mvec << 12) | (idx & (QW - 1))
                plsc.store_scatter(elist_v, [qsplat[q], pos], e, mask=mask)
                new.append(cnts[q] + jnp.sum(mi))
            return tuple(new)

        cnts = lax.fori_loop(
            0, M // L, compact_body,
            tuple(jnp.int32(0) for _ in range(NQ)))
        nwins = [(cnts[q] + (L - 1)) // L for q in range(NQ)]

        @pl.loop(0, SLABS, step=2)
        def _(si):
            for sp in range(2):
                slab = si + sp
                obuf = obufs[sp]
                osem = osems[sp]

                @pl.when(slab >= 2)
                def _():
                    pltpu.make_async_copy(obuf, odst(slab - 2), osem).wait()

                for q in range(NQ):
                    buf = bufs[q % 2]
                    isem = isems[q % 2]
                    t = slab * NQ + q
                    pltpu.make_async_copy(src(t), buf, isem).wait()

                    @pl.loop(0, nwins[q])
                    def _(jw):
                        j = pl.multiple_of(jw * L, L)
                        e = elist_v[q, pl.ds(j, L)]
                        nq = e & (QW - 1)
                        dest = e >> 12
                        maskw = (j + iota) < cnts[q]
                        for r in range(ROWS):
                            v = plsc.load_gather(buf, [rsplat[r], nq])
                            plsc.store_scatter(
                                obuf, [rsplat[r], dest], v, mask=maskw)

                    @pl.when(t + 2 < NT)
                    def _():
                        pltpu.async_copy(src(t + 2), buf, isem)

                pltpu.async_copy(obuf, odst(slab), osem)

        pltpu.make_async_copy(obuf_a, odst(SLABS - 2), osem_a).wait()
        pltpu.make_async_copy(obuf_b, odst(SLABS - 1), osem_b).wait()

    return k(feat, inds3)


def _sc_gather_xyz(xyz_flat, inds3):
    @functools.partial(
        pl.kernel,
        compiler_params=_compiler_params(),
        out_type=jax.ShapeDtypeStruct((B, TPB, XCH, 128), jnp.float32),
        mesh=_mesh(),
        scratch_types=[
            pltpu.VMEM((M // 128, 128), jnp.int32),
            pltpu.VMEM((XCH, 128), jnp.int32),
            pltpu.VMEM((XCH, 128), jnp.float32),
            pltpu.SemaphoreType.DMA,
        ],
    )
    def k(xyz_hbm, inds_hbm, oxyz_hbm, inds_v, widx_v, oxyz_v, xsem):
        wid = lax.axis_index("c") * NS + lax.axis_index("s")
        b = wid // TPB
        q_tile = wid % TPB
        iota = lax.iota(jnp.int32, L)

        pltpu.sync_copy(inds_hbm.at[b], inds_v)
        mbase = q_tile * MPT

        @pl.loop(0, MPT, step=L)
        def _(ml):
            mg = mbase + ml
            idx = inds_v[mg // 128, pl.ds(mg % 128, L)]
            g = idx + (b * 3) * N
            p0 = (ml + iota) * 3
            for j in range(3):
                p = p0 + j
                plsc.store_scatter(widx_v, [p // 128, p % 128], g + j * N)

        for ch in range(XCH):
            pltpu.async_copy(xyz_hbm.at[widx_v.at[ch]], oxyz_v.at[ch], xsem)
        for ch in range(XCH):
            pltpu.make_async_copy(
                xyz_hbm.at[widx_v.at[ch]], oxyz_v.at[ch], xsem).wait()
        pltpu.sync_copy(oxyz_v, oxyz_hbm.at[b, q_tile])

    return k(xyz_flat, inds3)


def kernel(xyz, features, sample_inds):
    inds3 = sample_inds.astype(jnp.int32).reshape(B, M // 128, 128)
    new_features = _sc_gather_features(features, inds3)
    xyz_flat = xyz.transpose(0, 2, 1).reshape(B * 3 * N)
    inds3x, _ = lax.optimization_barrier((inds3, new_features))
    oxyz = _sc_gather_xyz(xyz_flat, inds3x)
    new_xyz = oxyz.reshape(B, M, 3)
    return (new_xyz, new_features, sample_inds)


# speedup vs baseline: 2.8037x; 1.0606x over previous
import dataclasses
import functools

import jax
import jax.numpy as jnp
from jax import lax
from jax.experimental import pallas as pl
from jax.experimental.pallas import tpu as pltpu
from jax.experimental.pallas import tpu_sc as plsc

B, N, C, M = 8, 16384, 256, 2048
NC, NS = 2, 16
NW = NC * NS
TPB = NW // B
CPT = C // TPB
MPT = M // TPB
L = 16
ROWS = 8
SLABS = CPT // ROWS
NQ = 8
QW = N // NQ
QSH = 11
NB = 4
NT = SLABS * NQ
XCH = MPT * 3 // 128


def _compiler_params():
    cp = pltpu.CompilerParams()
    fields = pltpu.CompilerParams.__dataclass_fields__
    if "needs_layout_passes" in fields:
        cp = dataclasses.replace(cp, needs_layout_passes=False)
    return cp


def _mesh():
    return plsc.VectorSubcoreMesh(core_axis_name="c", subcore_axis_name="s")


def _sc_gather_features(feat, inds3):
    @functools.partial(
        pl.kernel,
        compiler_params=_compiler_params(),
        out_type=jax.ShapeDtypeStruct((B, C, M), jnp.float32),
        mesh=_mesh(),
        scratch_types=(
            [pltpu.VMEM((M // 128, 128), jnp.int32)]
            + [pltpu.VMEM((ROWS, QW), jnp.float32)] * NB
            + [pltpu.VMEM((ROWS, M), jnp.float32)] * 2
            + [pltpu.VMEM((NQ, M), jnp.int32)]
            + [pltpu.SemaphoreType.DMA] * NB
            + [pltpu.SemaphoreType.DMA] * 2
        ),
    )
    def k(feat_hbm, inds_hbm, ofeat_hbm,
          inds_v, buf_0, buf_1, buf_2, buf_3, obuf_a, obuf_b, elist_v,
          isem_0, isem_1, isem_2, isem_3, osem_a, osem_b):
        buf_a = buf_0
        buf_b = buf_1
        wid = lax.axis_index("c") * NS + lax.axis_index("s")
        b = wid // TPB
        q_tile = wid % TPB
        cbase = q_tile * CPT
        bufs = (buf_0, buf_1, buf_2, buf_3)
        obufs = (obuf_a, obuf_b)
        isems = (isem_0, isem_1, isem_2, isem_3)
        osems = (osem_a, osem_b)
        iota = lax.iota(jnp.int32, L)
        rsplat = [jnp.full((L,), r, jnp.int32) for r in range(ROWS)]
        qsplat = [jnp.full((L,), q, jnp.int32) for q in range(NQ)]

        def src(t):
            c0 = pl.multiple_of(cbase + (t // NQ) * ROWS, ROWS)
            n0 = pl.multiple_of((t % NQ) * QW, QW)
            return feat_hbm.at[b, pl.ds(c0, ROWS), pl.ds(n0, QW)]

        def odst(slab):
            c0 = pl.multiple_of(cbase + slab * ROWS, ROWS)
            return ofeat_hbm.at[b, pl.ds(c0, ROWS), :]

        pltpu.sync_copy(inds_hbm.at[b], inds_v)

        for t0 in range(NB):
            pltpu.async_copy(src(t0), bufs[t0], isems[t0])

        def compact_body(w, cnts):
            row = w // 8
            col = pl.multiple_of((w % 8) * L, L)
            idx = inds_v[row, pl.ds(col, L)]
            mvec = w * L + iota
            new = []
            for q in range(NQ):
                mask = (idx >> QSH) == q
                mi = mask.astype(jnp.int32)
                pos = cnts[q] + plsc.cumsum(mi) - 1
                e = (mvec << QSH) | (idx & (QW - 1))
                plsc.store_scatter(elist_v, [qsplat[q], pos], e, mask=mask)
                new.append(cnts[q] + jnp.sum(mi))
            return tuple(new)

        cnts = lax.fori_loop(
            0, M // L, compact_body,
            tuple(jnp.int32(0) for _ in range(NQ)))
        nwins = [(cnts[q] + (L - 1)) // L for q in range(NQ)]

        @pl.loop(0, SLABS, step=2)
        def _(si):
            for sp in range(2):
                slab = si + sp
                obuf = obufs[sp]
                osem = osems[sp]

                @pl.when(slab >= 2)
                def _():
                    pltpu.make_async_copy(obuf, odst(slab - 2), osem).wait()

                for q in range(NQ):
                    buf = bufs[q % NB]
                    isem = isems[q % NB]
                    t = slab * NQ + q
                    pltpu.make_async_copy(src(t), buf, isem).wait()

                    @pl.loop(0, nwins[q])
                    def _(jw):
                        j = pl.multiple_of(jw * L, L)
                        e = elist_v[q, pl.ds(j, L)]
                        nq = e & (QW - 1)
                        dest = e >> QSH
                        maskw = (j + iota) < cnts[q]
                        for r in range(ROWS):
                            v = plsc.load_gather(buf, [rsplat[r], nq])
                            plsc.store_scatter(
                                obuf, [rsplat[r], dest], v, mask=maskw)

                    @pl.when(t + NB < NT)
                    def _():
                        pltpu.async_copy(src(t + NB), buf, isem)

                pltpu.async_copy(obuf, odst(slab), osem)

        pltpu.make_async_copy(obuf_a, odst(SLABS - 2), osem_a).wait()
        pltpu.make_async_copy(obuf_b, odst(SLABS - 1), osem_b).wait()

    return k(feat, inds3)


def _sc_gather_xyz(xyz_flat, inds3):
    @functools.partial(
        pl.kernel,
        compiler_params=_compiler_params(),
        out_type=jax.ShapeDtypeStruct((B, TPB, XCH, 128), jnp.float32),
        mesh=_mesh(),
        scratch_types=[
            pltpu.VMEM((M // 128, 128), jnp.int32),
            pltpu.VMEM((XCH, 128), jnp.int32),
            pltpu.VMEM((XCH, 128), jnp.float32),
            pltpu.SemaphoreType.DMA,
        ],
    )
    def k(xyz_hbm, inds_hbm, oxyz_hbm, inds_v, widx_v, oxyz_v, xsem):
        wid = lax.axis_index("c") * NS + lax.axis_index("s")
        b = wid // TPB
        q_tile = wid % TPB
        iota = lax.iota(jnp.int32, L)

        pltpu.sync_copy(inds_hbm.at[b], inds_v)
        mbase = q_tile * MPT

        @pl.loop(0, MPT, step=L)
        def _(ml):
            mg = mbase + ml
            idx = inds_v[mg // 128, pl.ds(mg % 128, L)]
            g = idx + (b * 3) * N
            p0 = (ml + iota) * 3
            for j in range(3):
                p = p0 + j
                plsc.store_scatter(widx_v, [p // 128, p % 128], g + j * N)

        for ch in range(XCH):
            pltpu.async_copy(xyz_hbm.at[widx_v.at[ch]], oxyz_v.at[ch], xsem)
        for ch in range(XCH):
            pltpu.make_async_copy(
                xyz_hbm.at[widx_v.at[ch]], oxyz_v.at[ch], xsem).wait()
        pltpu.sync_copy(oxyz_v, oxyz_hbm.at[b, q_tile])

    return k(xyz_flat, inds3)


def kernel(xyz, features, sample_inds):
    inds3 = sample_inds.astype(jnp.int32).reshape(B, M // 128, 128)
    new_features = _sc_gather_features(features, inds3)
    xyz_flat = xyz.transpose(0, 2, 1).reshape(B * 3 * N)
    inds3x, _ = lax.optimization_barrier((inds3, new_features))
    oxyz = _sc_gather_xyz(xyz_flat, inds3x)
    new_xyz = oxyz.reshape(B, M, 3)
    return (new_xyz, new_features, sample_inds)
